# baseline (speedup 1.0000x reference)
import jax
import jax.numpy as jnp
from jax import lax
from jax.experimental import pallas as pl
from jax.experimental.pallas import tpu as pltpu
from jax.experimental.pallas import tpu_sc as plsc

S = 64.0
M = 0.4
H = 0.333
T_ALPHA = 0.01
EPS = 0.001

B = 1024
C = 100000
NC, NS, L = 2, 16, 16
NW = NC * NS
BPW = B // NW

G = B * C // L
GPW = G // NW
NGRP = 1250
NCHK = GPW // NGRP
NPAIR = NCHK // 2


def _clipn(v):
    return jnp.minimum(jnp.maximum(v, 0.001), 100.0)


def _sqrt16(x):
    tiny = 1e-20
    xc = jnp.maximum(x, tiny)
    i = plsc.bitcast(xc, jnp.int32)
    y = plsc.bitcast(0x5F3759DF - (i >> 1), jnp.float32)
    for _ in range(3):
        y = y * (1.5 - 0.5 * xc * y * y)
    r = xc * y
    r = 0.5 * (r + xc / r)
    return jnp.where(x <= tiny, 0.0, r)


def _sc_body(lg16_h, labels_h, norms_h, bm_h, bs_h,
             out16_h, nm_h, ns_h,
             lab_v, norms_v, idx_v, rows_v, marg_v, bm_v, bs_v,
             nm_v, ns_v, red_v, i0, i1, o0, o1,
             sem_g, s_i0, s_i1, s_o0, s_o1):
    wid = lax.axis_index("s") * NC + lax.axis_index("c")
    base = wid * BPW
    gbase = wid * GPW
    iota = lax.iota(jnp.int32, L)

    def _lanesum(vec):
        for k in (1, 2, 4, 8):
            red_v[...] = vec
            vec = vec + plsc.load_gather(red_v, [iota ^ k])
        return vec

    pltpu.sync_copy(labels_h.at[pl.ds(base, BPW)], lab_v)
    pltpu.sync_copy(norms_h, norms_v)
    pltpu.sync_copy(bm_h, bm_v)
    pltpu.sync_copy(bs_h, bs_v)

    def _sum_body(i, acc):
        return acc + _clipn(norms_v[pl.ds(i * L, L)])
    acc = lax.fori_loop(0, B // L, _sum_body, jnp.zeros((L,), jnp.float32))
    mean = _lanesum(acc) / B

    def _var_body(i, acc):
        d = _clipn(norms_v[pl.ds(i * L, L)]) - mean
        return acc + d * d
    acc2 = lax.fori_loop(0, B // L, _var_body, jnp.zeros((L,), jnp.float32))
    var = _lanesum(acc2) / (B - 1)
    std = _sqrt16(var)

    nm_vec = T_ALPHA * mean + (1.0 - T_ALPHA) * bm_v[...]
    ns_vec = T_ALPHA * std + (1.0 - T_ALPHA) * bs_v[...]
    nm_v[...] = nm_vec
    ns_v[...] = ns_vec

    for k in range(BPW // L):
        lab = lab_v[pl.ds(k * L, L)]
        rid = base + k * L + iota
        idx_v[pl.ds(k * L, L)] = (rid * C + lab) >> 4
    pltpu.async_copy(lg16_h.at[idx_v], rows_v, sem_g).wait()

    for k in range(BPW // L):
        lab = lab_v[pl.ds(k * L, L)]
        rid = base + k * L + iota
        off = (rid * C + lab) & 15
        t = plsc.load_gather(rows_v, [iota + k * L, off])
        n = _clipn(norms_v[pl.ds(base + k * L, L)])
        ms = jnp.clip((n - nm_vec) / (ns_vec + EPS) * H, -1.0, 1.0)
        a = M * ms
        a2 = a * a
        cos_a = 1.0 + a2 * (-0.5 + a2 * (1.0 / 24.0 + a2 * (
            -1.0 / 720.0 + a2 * (1.0 / 40320.0))))
        sin_a = a * (1.0 + a2 * (-1.0 / 6.0 + a2 * (
            1.0 / 120.0 + a2 * (-1.0 / 5040.0))))
        root = _sqrt16(1.0 - t * t)
        marg_v[pl.ds(k * L, L)] = (t * cos_a + root * sin_a - (M + a)) * S

    def _in_cp(k, buf, sem):
        return pltpu.make_async_copy(
            lg16_h.at[pl.ds(gbase + k * NGRP, NGRP)], buf, sem)

    def _out_cp(k, buf, sem):
        return pltpu.make_async_copy(
            buf, out16_h.at[pl.ds(gbase + k * NGRP, NGRP)], sem)

    def _scale(ibuf, obuf):
        @plsc.parallel_loop(0, NGRP, unroll=5)
        def _(i):
            obuf[i] = ibuf[i] * S

    _in_cp(0, i0, s_i0).start()
    _in_cp(1, i1, s_i1).start()

    def _pair(p, carry):
        k0 = 2 * p
        _in_cp(k0, i0, s_i0).wait()

        @pl.when(p > 0)
        def _():
            _out_cp(k0, o0, s_o0).wait()
        _scale(i0, o0)
        _out_cp(k0, o0, s_o0).start()

        @pl.when(p < NPAIR - 1)
        def _():
            _in_cp(k0 + 2, i0, s_i0).start()

        _in_cp(k0 + 1, i1, s_i1).wait()

        @pl.when(p > 0)
        def _():
            _out_cp(k0 + 1, o1, s_o1).wait()
        _scale(i1, o1)
        _out_cp(k0 + 1, o1, s_o1).start()

        @pl.when(p < NPAIR - 1)
        def _():
            _in_cp(k0 + 3, i1, s_i1).start()
        return carry

    lax.fori_loop(0, NPAIR, _pair, 0)
    _out_cp(NCHK - 2, o0, s_o0).wait()
    _out_cp(NCHK - 1, o1, s_o1).wait()

    pltpu.async_copy(out16_h.at[idx_v], rows_v, sem_g).wait()
    for k in range(BPW // L):
        lab = lab_v[pl.ds(k * L, L)]
        rid = base + k * L + iota
        off = (rid * C + lab) & 15
        plsc.store_scatter(rows_v, [iota + k * L, off],
                           marg_v[pl.ds(k * L, L)])
    pltpu.async_copy(rows_v, out16_h.at[idx_v], sem_g).wait()

    @pl.when(wid == 0)
    def _():
        pltpu.sync_copy(nm_v, nm_h)
        pltpu.sync_copy(ns_v, ns_h)


_sc_run = pl.kernel(
    _sc_body,
    out_type=[
        jax.ShapeDtypeStruct((G, L), jnp.float32),
        jax.ShapeDtypeStruct((L,), jnp.float32),
        jax.ShapeDtypeStruct((L,), jnp.float32),
    ],
    mesh=plsc.VectorSubcoreMesh(core_axis_name="c", subcore_axis_name="s"),
    compiler_params=pltpu.CompilerParams(
        needs_layout_passes=False, use_tc_tiling_on_sc=False),
    scratch_types=[
        pltpu.VMEM((BPW,), jnp.int32),
        pltpu.VMEM((B,), jnp.float32),
        pltpu.VMEM((BPW,), jnp.int32),
        pltpu.VMEM((BPW, L), jnp.float32),
        pltpu.VMEM((BPW,), jnp.float32),
        pltpu.VMEM((L,), jnp.float32),
        pltpu.VMEM((L,), jnp.float32),
        pltpu.VMEM((L,), jnp.float32),
        pltpu.VMEM((L,), jnp.float32),
        pltpu.VMEM((L,), jnp.float32),
        pltpu.VMEM((NGRP, L), jnp.float32),
        pltpu.VMEM((NGRP, L), jnp.float32),
        pltpu.VMEM((NGRP, L), jnp.float32),
        pltpu.VMEM((NGRP, L), jnp.float32),
        pltpu.SemaphoreType.DMA,
        pltpu.SemaphoreType.DMA,
        pltpu.SemaphoreType.DMA,
        pltpu.SemaphoreType.DMA,
        pltpu.SemaphoreType.DMA,
    ],
)


def kernel(logits, labels, norms, batch_mean, batch_std):
    lg16 = logits.reshape(G, L)
    bm16 = jnp.broadcast_to(batch_mean, (L,))
    bs16 = jnp.broadcast_to(batch_std, (L,))
    out16, nm16, ns16 = _sc_run(lg16, labels, norms.reshape(B), bm16, bs16)
    return out16.reshape(B, C), nm16[:1], ns16[:1]


# speedup vs baseline: 1.2958x; 1.2958x over previous
import jax
import jax.numpy as jnp
from jax import lax
from jax.experimental import pallas as pl
from jax.experimental.pallas import tpu as pltpu
from jax.experimental.pallas import tpu_sc as plsc

S = 64.0
M = 0.4
H = 0.333
T_ALPHA = 0.01
EPS = 0.001

B = 1024
C = 100000
NC, NS, L = 2, 16, 16
NW = NC * NS
BPW = B // NW

RB = 8
NBUF = 4
NSTEP = B // RB // NBUF


def _clipn(v):
    return jnp.minimum(jnp.maximum(v, 0.001), 100.0)


def _sqrt16(x):
    tiny = 1e-20
    xc = jnp.maximum(x, tiny)
    i = plsc.bitcast(xc, jnp.int32)
    y = plsc.bitcast(0x5F3759DF - (i >> 1), jnp.float32)
    for _ in range(3):
        y = y * (1.5 - 0.5 * xc * y * y)
    r = xc * y
    r = 0.5 * (r + xc / r)
    return jnp.where(x <= tiny, 0.0, r)


def _sc_body(lg16_h, labels_h, norms_h, bm_h, bs_h,
             marg_h, nm_h, ns_h,
             lab_v, norms_v, idx_v, rows_v, marg_v, bm_v, bs_v,
             nm_v, ns_v, red_v, sem):
    wid = lax.axis_index("s") * NC + lax.axis_index("c")
    base = wid * BPW
    iota = lax.iota(jnp.int32, L)

    def _lanesum(vec):
        for k in (1, 2, 4, 8):
            red_v[...] = vec
            vec = vec + plsc.load_gather(red_v, [iota ^ k])
        return vec

    pltpu.sync_copy(labels_h.at[pl.ds(base, BPW)], lab_v)
    pltpu.sync_copy(norms_h, norms_v)
    pltpu.sync_copy(bm_h, bm_v)
    pltpu.sync_copy(bs_h, bs_v)

    def _sum_body(i, acc):
        return acc + _clipn(norms_v[pl.ds(i * L, L)])
    acc = lax.fori_loop(0, B // L, _sum_body, jnp.zeros((L,), jnp.float32))
    mean = _lanesum(acc) / B

    def _var_body(i, acc):
        d = _clipn(norms_v[pl.ds(i * L, L)]) - mean
        return acc + d * d
    acc2 = lax.fori_loop(0, B // L, _var_body, jnp.zeros((L,), jnp.float32))
    var = _lanesum(acc2) / (B - 1)
    std = _sqrt16(var)

    nm_vec = T_ALPHA * mean + (1.0 - T_ALPHA) * bm_v[...]
    ns_vec = T_ALPHA * std + (1.0 - T_ALPHA) * bs_v[...]

    for k in range(BPW // L):
        lab = lab_v[pl.ds(k * L, L)]
        rid = base + k * L + iota
        idx_v[pl.ds(k * L, L)] = (rid * C + lab) >> 4
    pltpu.async_copy(lg16_h.at[idx_v], rows_v, sem).wait()

    for k in range(BPW // L):
        lab = lab_v[pl.ds(k * L, L)]
        rid = base + k * L + iota
        off = (rid * C + lab) & 15
        t = plsc.load_gather(rows_v, [iota + k * L, off])
        n = _clipn(norms_v[pl.ds(base + k * L, L)])
        ms = jnp.clip((n - nm_vec) / (ns_vec + EPS) * H, -1.0, 1.0)
        a = M * ms
        a2 = a * a
        cos_a = 1.0 + a2 * (-0.5 + a2 * (1.0 / 24.0 + a2 * (
            -1.0 / 720.0 + a2 * (1.0 / 40320.0))))
        sin_a = a * (1.0 + a2 * (-1.0 / 6.0 + a2 * (
            1.0 / 120.0 + a2 * (-1.0 / 5040.0))))
        root = _sqrt16(1.0 - t * t)
        marg_v[pl.ds(k * L, L)] = (t * cos_a + root * sin_a - (M + a)) * S

    pltpu.sync_copy(marg_v, marg_h.at[pl.ds(base, BPW)])

    nm_v[...] = nm_vec
    ns_v[...] = ns_vec

    @pl.when(wid == 0)
    def _():
        pltpu.sync_copy(nm_v, nm_h)
        pltpu.sync_copy(ns_v, ns_h)


_sc_prep = pl.kernel(
    _sc_body,
    out_type=[
        jax.ShapeDtypeStruct((B,), jnp.float32),
        jax.ShapeDtypeStruct((L,), jnp.float32),
        jax.ShapeDtypeStruct((L,), jnp.float32),
    ],
    mesh=plsc.VectorSubcoreMesh(core_axis_name="c", subcore_axis_name="s"),
    compiler_params=pltpu.CompilerParams(
        needs_layout_passes=False, use_tc_tiling_on_sc=False),
    scratch_types=[
        pltpu.VMEM((BPW,), jnp.int32),
        pltpu.VMEM((B,), jnp.float32),
        pltpu.VMEM((BPW,), jnp.int32),
        pltpu.VMEM((BPW, L), jnp.float32),
        pltpu.VMEM((BPW,), jnp.float32),
        pltpu.VMEM((L,), jnp.float32),
        pltpu.VMEM((L,), jnp.float32),
        pltpu.VMEM((L,), jnp.float32),
        pltpu.VMEM((L,), jnp.float32),
        pltpu.VMEM((L,), jnp.float32),
        pltpu.SemaphoreType.DMA,
    ],
)


def _tc_body(lab_ref, marg_ref, lg_ref, o_ref, *bufs):
    ibufs, obufs = bufs[0:NBUF], bufs[NBUF:2 * NBUF]
    isems, osems = bufs[2 * NBUF:3 * NBUF], bufs[3 * NBUF:4 * NBUF]
    j = pl.program_id(0)

    def in_cp(c, buf, sem):
        return pltpu.make_async_copy(
            lg_ref.at[pl.ds(c * RB, RB), :], buf, sem)

    def out_cp(c, buf, sem):
        return pltpu.make_async_copy(
            buf, o_ref.at[pl.ds(c * RB, RB), :], sem)

    @pl.when(j == 0)
    def _():
        for b in range(NBUF):
            in_cp(b, ibufs[b], isems[b]).start()

    for b in range(NBUF):
        c = j * NBUF + b
        in_cp(c, ibufs[b], isems[b]).wait()

        @pl.when(j > 0)
        def _(b=b, c=c):
            out_cp(c - NBUF, obufs[b], osems[b]).wait()

        lab = lab_ref[pl.ds(c * RB, RB), :]
        mg = marg_ref[pl.ds(c * RB, RB), :]
        cols = lax.broadcasted_iota(jnp.int32, (RB, C), 1)
        obufs[b][...] = jnp.where(cols == lab, mg, ibufs[b][...] * S)
        out_cp(c, obufs[b], osems[b]).start()

        @pl.when(j < NSTEP - 1)
        def _(b=b, c=c):
            in_cp(c + NBUF, ibufs[b], isems[b]).start()

    @pl.when(j == NSTEP - 1)
    def _():
        for b in range(NBUF):
            out_cp(j * NBUF + b, obufs[b], osems[b]).wait()


def _tc_scale(logits, labels2d, margins2d):
    return pl.pallas_call(
        _tc_body,
        grid=(NSTEP,),
        in_specs=[
            pl.BlockSpec((B, 1), lambda j: (0, 0)),
            pl.BlockSpec((B, 1), lambda j: (0, 0)),
            pl.BlockSpec(memory_space=pl.ANY),
        ],
        out_specs=pl.BlockSpec(memory_space=pl.ANY),
        out_shape=jax.ShapeDtypeStruct((B, C), jnp.float32),
        scratch_shapes=(
            [pltpu.VMEM((RB, C), jnp.float32)] * (2 * NBUF)
            + [pltpu.SemaphoreType.DMA] * (2 * NBUF)
        ),
        compiler_params=pltpu.CompilerParams(
            dimension_semantics=("arbitrary",)),
    )(labels2d, margins2d, logits)


def kernel(logits, labels, norms, batch_mean, batch_std):
    lg16 = logits.reshape(B * C // L, L)
    bm16 = jnp.broadcast_to(batch_mean, (L,))
    bs16 = jnp.broadcast_to(batch_std, (L,))
    margins, nm16, ns16 = _sc_prep(
        lg16, labels, norms.reshape(B), bm16, bs16)
    out = _tc_scale(logits, labels.reshape(B, 1), margins.reshape(B, 1))
    return out, nm16[:1], ns16[:1]


# speedup vs baseline: 2.0506x; 1.5825x over previous
import jax
import jax.numpy as jnp
from jax import lax
from jax.experimental import pallas as pl
from jax.experimental.pallas import tpu as pltpu
from jax.experimental.pallas import tpu_sc as plsc

S = 64.0
M = 0.4
H = 0.333
T_ALPHA = 0.01
EPS = 0.001

B = 1024
C = 100000
NC, NS, L = 2, 16, 16
NW = NC * NS
BPW = B // NW

RB = 8
NBUF = 4
NSTEP = B // RB // NBUF


def _clipn(v):
    return jnp.minimum(jnp.maximum(v, 0.001), 100.0)


def _sqrt16(x):
    tiny = 1e-20
    xc = jnp.maximum(x, tiny)
    i = plsc.bitcast(xc, jnp.int32)
    y = plsc.bitcast(0x5F3759DF - (i >> 1), jnp.float32)
    for _ in range(3):
        y = y * (1.5 - 0.5 * xc * y * y)
    r = xc * y
    r = 0.5 * (r + xc / r)
    return jnp.where(x <= tiny, 0.0, r)


def _sc_body(norms_h, bm_h, bs_h,
             ca_h, sa_h, off_h, nm_h, ns_h,
             norms_v, bm_v, bs_v, nm_v, ns_v, red_v,
             ca_v, sa_v, off_v):
    wid = lax.axis_index("s") * NC + lax.axis_index("c")
    base = wid * BPW
    iota = lax.iota(jnp.int32, L)

    def _lanesum(vec):
        for k in (1, 2, 4, 8):
            red_v[...] = vec
            vec = vec + plsc.load_gather(red_v, [iota ^ k])
        return vec

    pltpu.sync_copy(norms_h, norms_v)
    pltpu.sync_copy(bm_h, bm_v)
    pltpu.sync_copy(bs_h, bs_v)

    def _sum_body(i, acc):
        return acc + _clipn(norms_v[pl.ds(i * L, L)])
    acc = lax.fori_loop(0, B // L, _sum_body, jnp.zeros((L,), jnp.float32))
    mean = _lanesum(acc) / B

    def _var_body(i, acc):
        d = _clipn(norms_v[pl.ds(i * L, L)]) - mean
        return acc + d * d
    acc2 = lax.fori_loop(0, B // L, _var_body, jnp.zeros((L,), jnp.float32))
    var = _lanesum(acc2) / (B - 1)
    std = _sqrt16(var)

    nm_vec = T_ALPHA * mean + (1.0 - T_ALPHA) * bm_v[...]
    ns_vec = T_ALPHA * std + (1.0 - T_ALPHA) * bs_v[...]
    nm_v[...] = nm_vec
    ns_v[...] = ns_vec

    for k in range(BPW // L):
        n = _clipn(norms_v[pl.ds(base + k * L, L)])
        ms = jnp.clip((n - nm_vec) / (ns_vec + EPS) * H, -1.0, 1.0)
        a = M * ms
        a2 = a * a
        cos_a = 1.0 + a2 * (-0.5 + a2 * (1.0 / 24.0 + a2 * (
            -1.0 / 720.0 + a2 * (1.0 / 40320.0))))
        sin_a = a * (1.0 + a2 * (-1.0 / 6.0 + a2 * (
            1.0 / 120.0 + a2 * (-1.0 / 5040.0))))
        ca_v[pl.ds(k * L, L)] = cos_a
        sa_v[pl.ds(k * L, L)] = sin_a
        off_v[pl.ds(k * L, L)] = M + a

    pltpu.sync_copy(ca_v, ca_h.at[pl.ds(base, BPW)])
    pltpu.sync_copy(sa_v, sa_h.at[pl.ds(base, BPW)])
    pltpu.sync_copy(off_v, off_h.at[pl.ds(base, BPW)])

    @pl.when(wid == 0)
    def _():
        pltpu.sync_copy(nm_v, nm_h)
        pltpu.sync_copy(ns_v, ns_h)


_sc_prep = pl.kernel(
    _sc_body,
    out_type=[
        jax.ShapeDtypeStruct((B,), jnp.float32),
        jax.ShapeDtypeStruct((B,), jnp.float32),
        jax.ShapeDtypeStruct((B,), jnp.float32),
        jax.ShapeDtypeStruct((L,), jnp.float32),
        jax.ShapeDtypeStruct((L,), jnp.float32),
    ],
    mesh=plsc.VectorSubcoreMesh(core_axis_name="c", subcore_axis_name="s"),
    compiler_params=pltpu.CompilerParams(
        needs_layout_passes=False, use_tc_tiling_on_sc=False),
    scratch_types=[
        pltpu.VMEM((B,), jnp.float32),
        pltpu.VMEM((L,), jnp.float32),
        pltpu.VMEM((L,), jnp.float32),
        pltpu.VMEM((L,), jnp.float32),
        pltpu.VMEM((L,), jnp.float32),
        pltpu.VMEM((L,), jnp.float32),
        pltpu.VMEM((BPW,), jnp.float32),
        pltpu.VMEM((BPW,), jnp.float32),
        pltpu.VMEM((BPW,), jnp.float32),
    ],
)


def _tc_body(lab_ref, ca_ref, sa_ref, off_ref, lg_ref, o_ref, *bufs):
    ibufs, obufs = bufs[0:NBUF], bufs[NBUF:2 * NBUF]
    isems, osems = bufs[2 * NBUF:3 * NBUF], bufs[3 * NBUF:4 * NBUF]
    j = pl.program_id(0)

    def in_cp(c, buf, sem):
        return pltpu.make_async_copy(
            lg_ref.at[pl.ds(c * RB, RB), :], buf, sem)

    def out_cp(c, buf, sem):
        return pltpu.make_async_copy(
            buf, o_ref.at[pl.ds(c * RB, RB), :], sem)

    @pl.when(j == 0)
    def _():
        for b in range(NBUF):
            in_cp(b, ibufs[b], isems[b]).start()

    for b in range(NBUF):
        c = j * NBUF + b
        in_cp(c, ibufs[b], isems[b]).wait()

        @pl.when(j > 0)
        def _(b=b, c=c):
            out_cp(c - NBUF, obufs[b], osems[b]).wait()

        x = ibufs[b][...]
        lab = lab_ref[pl.ds(c * RB, RB), :]
        ca = ca_ref[pl.ds(c * RB, RB), :]
        sa = sa_ref[pl.ds(c * RB, RB), :]
        off = off_ref[pl.ds(c * RB, RB), :]
        cols = lax.broadcasted_iota(jnp.int32, (RB, C), 1)
        mask = cols == lab
        t = jnp.sum(jnp.where(mask, x, 0.0), axis=1, keepdims=True)
        root = jnp.sqrt(jnp.maximum(1.0 - t * t, 0.0))
        g = (t * ca + root * sa - off) * S
        obufs[b][...] = jnp.where(mask, g, x * S)
        out_cp(c, obufs[b], osems[b]).start()

        @pl.when(j < NSTEP - 1)
        def _(b=b, c=c):
            in_cp(c + NBUF, ibufs[b], isems[b]).start()

    @pl.when(j == NSTEP - 1)
    def _():
        for b in range(NBUF):
            out_cp(j * NBUF + b, obufs[b], osems[b]).wait()


def _tc_scale(logits, labels2d, ca2d, sa2d, off2d):
    return pl.pallas_call(
        _tc_body,
        grid=(NSTEP,),
        in_specs=[
            pl.BlockSpec((B, 1), lambda j: (0, 0)),
            pl.BlockSpec((B, 1), lambda j: (0, 0)),
            pl.BlockSpec((B, 1), lambda j: (0, 0)),
            pl.BlockSpec((B, 1), lambda j: (0, 0)),
            pl.BlockSpec(memory_space=pl.ANY),
        ],
        out_specs=pl.BlockSpec(memory_space=pl.ANY),
        out_shape=jax.ShapeDtypeStruct((B, C), jnp.float32),
        scratch_shapes=(
            [pltpu.VMEM((RB, C), jnp.float32)] * (2 * NBUF)
            + [pltpu.SemaphoreType.DMA] * (2 * NBUF)
        ),
        compiler_params=pltpu.CompilerParams(
            dimension_semantics=("arbitrary",)),
    )(labels2d, ca2d, sa2d, off2d, logits)


def kernel(logits, labels, norms, batch_mean, batch_std):
    bm16 = jnp.broadcast_to(batch_mean, (L,))
    bs16 = jnp.broadcast_to(batch_std, (L,))
    ca, sa, off, nm16, ns16 = _sc_prep(norms.reshape(B), bm16, bs16)
    out = _tc_scale(logits, labels.reshape(B, 1), ca.reshape(B, 1),
                    sa.reshape(B, 1), off.reshape(B, 1))
    return out, nm16[:1], ns16[:1]


# speedup vs baseline: 7.3226x; 3.5709x over previous
import jax
import jax.numpy as jnp
from jax import lax
from jax.experimental import pallas as pl
from jax.experimental.pallas import tpu as pltpu
from jax.experimental.pallas import tpu_sc as plsc

S = 64.0
M = 0.4
H = 0.333
T_ALPHA = 0.01
EPS = 0.001

B = 1024
C = 100000
NC, NS, L = 2, 16, 16
NW = NC * NS
BPW = B // NW

CH4 = 125
NBUF = 4
NSTEP = C // 8 // CH4 // NBUF


def _clipn(v):
    return jnp.minimum(jnp.maximum(v, 0.001), 100.0)


def _sqrt16(x):
    tiny = 1e-20
    xc = jnp.maximum(x, tiny)
    i = plsc.bitcast(xc, jnp.int32)
    y = plsc.bitcast(0x5F3759DF - (i >> 1), jnp.float32)
    for _ in range(3):
        y = y * (1.5 - 0.5 * xc * y * y)
    r = xc * y
    r = 0.5 * (r + xc / r)
    return jnp.where(x <= tiny, 0.0, r)


def _sc_body(norms_h, bm_h, bs_h,
             ca_h, sa_h, off_h, nm_h, ns_h,
             norms_v, bm_v, bs_v, nm_v, ns_v, red_v,
             ca_v, sa_v, off_v):
    wid = lax.axis_index("s") * NC + lax.axis_index("c")
    base = wid * BPW
    iota = lax.iota(jnp.int32, L)

    def _lanesum(vec):
        for k in (1, 2, 4, 8):
            red_v[...] = vec
            vec = vec + plsc.load_gather(red_v, [iota ^ k])
        return vec

    pltpu.sync_copy(norms_h, norms_v)
    pltpu.sync_copy(bm_h, bm_v)
    pltpu.sync_copy(bs_h, bs_v)

    def _sum_body(i, acc):
        return acc + _clipn(norms_v[pl.ds(i * L, L)])
    acc = lax.fori_loop(0, B // L, _sum_body, jnp.zeros((L,), jnp.float32))
    mean = _lanesum(acc) / B

    def _var_body(i, acc):
        d = _clipn(norms_v[pl.ds(i * L, L)]) - mean
        return acc + d * d
    acc2 = lax.fori_loop(0, B // L, _var_body, jnp.zeros((L,), jnp.float32))
    var = _lanesum(acc2) / (B - 1)
    std = _sqrt16(var)

    nm_vec = T_ALPHA * mean + (1.0 - T_ALPHA) * bm_v[...]
    ns_vec = T_ALPHA * std + (1.0 - T_ALPHA) * bs_v[...]
    nm_v[...] = nm_vec
    ns_v[...] = ns_vec

    for k in range(BPW // L):
        n = _clipn(norms_v[pl.ds(base + k * L, L)])
        ms = jnp.clip((n - nm_vec) / (ns_vec + EPS) * H, -1.0, 1.0)
        a = M * ms
        a2 = a * a
        cos_a = 1.0 + a2 * (-0.5 + a2 * (1.0 / 24.0 + a2 * (
            -1.0 / 720.0 + a2 * (1.0 / 40320.0))))
        sin_a = a * (1.0 + a2 * (-1.0 / 6.0 + a2 * (
            1.0 / 120.0 + a2 * (-1.0 / 5040.0))))
        ca_v[pl.ds(k * L, L)] = cos_a
        sa_v[pl.ds(k * L, L)] = sin_a
        off_v[pl.ds(k * L, L)] = M + a

    pltpu.sync_copy(ca_v, ca_h.at[pl.ds(base, BPW)])
    pltpu.sync_copy(sa_v, sa_h.at[pl.ds(base, BPW)])
    pltpu.sync_copy(off_v, off_h.at[pl.ds(base, BPW)])

    @pl.when(wid == 0)
    def _():
        pltpu.sync_copy(nm_v, nm_h)
        pltpu.sync_copy(ns_v, ns_h)


_sc_prep = pl.kernel(
    _sc_body,
    out_type=[
        jax.ShapeDtypeStruct((B,), jnp.float32),
        jax.ShapeDtypeStruct((B,), jnp.float32),
        jax.ShapeDtypeStruct((B,), jnp.float32),
        jax.ShapeDtypeStruct((L,), jnp.float32),
        jax.ShapeDtypeStruct((L,), jnp.float32),
    ],
    mesh=plsc.VectorSubcoreMesh(core_axis_name="c", subcore_axis_name="s"),
    compiler_params=pltpu.CompilerParams(
        needs_layout_passes=False, use_tc_tiling_on_sc=False),
    scratch_types=[
        pltpu.VMEM((B,), jnp.float32),
        pltpu.VMEM((L,), jnp.float32),
        pltpu.VMEM((L,), jnp.float32),
        pltpu.VMEM((L,), jnp.float32),
        pltpu.VMEM((L,), jnp.float32),
        pltpu.VMEM((L,), jnp.float32),
        pltpu.VMEM((BPW,), jnp.float32),
        pltpu.VMEM((BPW,), jnp.float32),
        pltpu.VMEM((BPW,), jnp.float32),
    ],
)


def _tc_body(lab_ref, ca_ref, sa_ref, off_ref, lg_ref, o_ref, *bufs):
    ibufs, obufs = bufs[0:NBUF], bufs[NBUF:2 * NBUF]
    isems, osems = bufs[2 * NBUF:3 * NBUF], bufs[3 * NBUF:4 * NBUF]
    j = pl.program_id(0)

    def in_cp(c, buf, sem):
        return pltpu.make_async_copy(
            lg_ref.at[pl.ds(c * CH4, CH4), :, :, :], buf, sem)

    def out_cp(c, buf, sem):
        return pltpu.make_async_copy(
            buf, o_ref.at[pl.ds(c * CH4, CH4), :, :, :], sem)

    @pl.when(j == 0)
    def _():
        for b in range(NBUF):
            in_cp(b, ibufs[b], isems[b]).start()

    for b in range(NBUF):
        c = j * NBUF + b
        in_cp(c, ibufs[b], isems[b]).wait()

        @pl.when(j > 0)
        def _(b=b, c=c):
            out_cp(c - NBUF, obufs[b], osems[b]).wait()

        x = ibufs[b][...]
        lab = lab_ref[...].reshape(1, B // 128, 1, 128)
        ca = ca_ref[...].reshape(1, B // 128, 1, 128)
        sa = sa_ref[...].reshape(1, B // 128, 1, 128)
        off = off_ref[...].reshape(1, B // 128, 1, 128)
        ci = lax.broadcasted_iota(jnp.int32, (CH4, 1, 8, 1), 0)
        ck = lax.broadcasted_iota(jnp.int32, (CH4, 1, 8, 1), 2)
        cls = (c * CH4 + ci) * 8 + ck
        mask = cls == lab
        t = jnp.sum(jnp.where(mask, x, 0.0), axis=0, keepdims=True)
        t = jnp.sum(t, axis=2, keepdims=True)
        root = jnp.sqrt(jnp.maximum(1.0 - t * t, 0.0))
        g = (t * ca + root * sa - off) * S
        obufs[b][...] = jnp.where(mask, g, x * S)
        out_cp(c, obufs[b], osems[b]).start()

        @pl.when(j < NSTEP - 1)
        def _(b=b, c=c):
            in_cp(c + NBUF, ibufs[b], isems[b]).start()

    @pl.when(j == NSTEP - 1)
    def _():
        for b in range(NBUF):
            out_cp(j * NBUF + b, obufs[b], osems[b]).wait()


def _tc_scale(lg4, labels2d, ca2d, sa2d, off2d):
    return pl.pallas_call(
        _tc_body,
        grid=(NSTEP,),
        in_specs=[
            pl.BlockSpec((B // 128, 128), lambda j: (0, 0)),
            pl.BlockSpec((B // 128, 128), lambda j: (0, 0)),
            pl.BlockSpec((B // 128, 128), lambda j: (0, 0)),
            pl.BlockSpec((B // 128, 128), lambda j: (0, 0)),
            pl.BlockSpec(memory_space=pl.ANY),
        ],
        out_specs=pl.BlockSpec(memory_space=pl.ANY),
        out_shape=jax.ShapeDtypeStruct((C // 8, B // 128, 8, 128),
                                       jnp.float32),
        scratch_shapes=(
            [pltpu.VMEM((CH4, B // 128, 8, 128), jnp.float32)] * (2 * NBUF)
            + [pltpu.SemaphoreType.DMA] * (2 * NBUF)
        ),
        compiler_params=pltpu.CompilerParams(
            dimension_semantics=("arbitrary",),
            vmem_limit_bytes=100 * 1024 * 1024),
    )(labels2d, ca2d, sa2d, off2d, lg4)


def kernel(logits, labels, norms, batch_mean, batch_std):
    bm16 = jnp.broadcast_to(batch_mean, (L,))
    bs16 = jnp.broadcast_to(batch_std, (L,))
    ca, sa, off, nm16, ns16 = _sc_prep(norms.reshape(B), bm16, bs16)
    lg4 = logits.T.reshape(C // 8, 8, B // 128, 128).transpose(0, 2, 1, 3)
    out4 = _tc_scale(lg4, labels.reshape(B // 128, 128),
                     ca.reshape(B // 128, 128), sa.reshape(B // 128, 128),
                     off.reshape(B // 128, 128))
    out = out4.transpose(0, 2, 1, 3).reshape(C, B).T
    return out, nm16[:1], ns16[:1]


# speedup vs baseline: 7.3240x; 1.0002x over previous
import jax
import jax.numpy as jnp
from jax import lax
from jax.experimental import pallas as pl
from jax.experimental.pallas import tpu as pltpu
from jax.experimental.pallas import tpu_sc as plsc

S = 64.0
M = 0.4
H = 0.333
T_ALPHA = 0.01
EPS = 0.001

B = 1024
C = 100000
NC, NS, L = 2, 16, 16
NW = NC * NS
BPW = B // NW

CH4 = 125
NBUF = 5
NSTEP = C // 8 // CH4 // NBUF


def _clipn(v):
    return jnp.minimum(jnp.maximum(v, 0.001), 100.0)


def _sqrt16(x):
    tiny = 1e-20
    xc = jnp.maximum(x, tiny)
    i = plsc.bitcast(xc, jnp.int32)
    y = plsc.bitcast(0x5F3759DF - (i >> 1), jnp.float32)
    for _ in range(3):
        y = y * (1.5 - 0.5 * xc * y * y)
    r = xc * y
    r = 0.5 * (r + xc / r)
    return jnp.where(x <= tiny, 0.0, r)


def _sc_body(norms_h, bm_h, bs_h,
             ca_h, sa_h, off_h, nm_h, ns_h,
             norms_v, bm_v, bs_v, nm_v, ns_v, red_v,
             ca_v, sa_v, off_v):
    wid = lax.axis_index("s") * NC + lax.axis_index("c")
    base = wid * BPW
    iota = lax.iota(jnp.int32, L)

    def _lanesum(vec):
        for k in (1, 2, 4, 8):
            red_v[...] = vec
            vec = vec + plsc.load_gather(red_v, [iota ^ k])
        return vec

    pltpu.sync_copy(norms_h, norms_v)
    pltpu.sync_copy(bm_h, bm_v)
    pltpu.sync_copy(bs_h, bs_v)

    def _sum_body(i, acc):
        return acc + _clipn(norms_v[pl.ds(i * L, L)])
    acc = lax.fori_loop(0, B // L, _sum_body, jnp.zeros((L,), jnp.float32))
    mean = _lanesum(acc) / B

    def _var_body(i, acc):
        d = _clipn(norms_v[pl.ds(i * L, L)]) - mean
        return acc + d * d
    acc2 = lax.fori_loop(0, B // L, _var_body, jnp.zeros((L,), jnp.float32))
    var = _lanesum(acc2) / (B - 1)
    std = _sqrt16(var)

    nm_vec = T_ALPHA * mean + (1.0 - T_ALPHA) * bm_v[...]
    ns_vec = T_ALPHA * std + (1.0 - T_ALPHA) * bs_v[...]
    nm_v[...] = nm_vec
    ns_v[...] = ns_vec

    for k in range(BPW // L):
        n = _clipn(norms_v[pl.ds(base + k * L, L)])
        ms = jnp.clip((n - nm_vec) / (ns_vec + EPS) * H, -1.0, 1.0)
        a = M * ms
        a2 = a * a
        cos_a = 1.0 + a2 * (-0.5 + a2 * (1.0 / 24.0 + a2 * (
            -1.0 / 720.0 + a2 * (1.0 / 40320.0))))
        sin_a = a * (1.0 + a2 * (-1.0 / 6.0 + a2 * (
            1.0 / 120.0 + a2 * (-1.0 / 5040.0))))
        ca_v[pl.ds(k * L, L)] = cos_a
        sa_v[pl.ds(k * L, L)] = sin_a
        off_v[pl.ds(k * L, L)] = M + a

    pltpu.sync_copy(ca_v, ca_h.at[pl.ds(base, BPW)])
    pltpu.sync_copy(sa_v, sa_h.at[pl.ds(base, BPW)])
    pltpu.sync_copy(off_v, off_h.at[pl.ds(base, BPW)])

    @pl.when(wid == 0)
    def _():
        pltpu.sync_copy(nm_v, nm_h)
        pltpu.sync_copy(ns_v, ns_h)


_sc_prep = pl.kernel(
    _sc_body,
    out_type=[
        jax.ShapeDtypeStruct((B,), jnp.float32),
        jax.ShapeDtypeStruct((B,), jnp.float32),
        jax.ShapeDtypeStruct((B,), jnp.float32),
        jax.ShapeDtypeStruct((L,), jnp.float32),
        jax.ShapeDtypeStruct((L,), jnp.float32),
    ],
    mesh=plsc.VectorSubcoreMesh(core_axis_name="c", subcore_axis_name="s"),
    compiler_params=pltpu.CompilerParams(
        needs_layout_passes=False, use_tc_tiling_on_sc=False),
    scratch_types=[
        pltpu.VMEM((B,), jnp.float32),
        pltpu.VMEM((L,), jnp.float32),
        pltpu.VMEM((L,), jnp.float32),
        pltpu.VMEM((L,), jnp.float32),
        pltpu.VMEM((L,), jnp.float32),
        pltpu.VMEM((L,), jnp.float32),
        pltpu.VMEM((BPW,), jnp.float32),
        pltpu.VMEM((BPW,), jnp.float32),
        pltpu.VMEM((BPW,), jnp.float32),
    ],
)


def _tc_body(lab_ref, ca_ref, sa_ref, off_ref, lg_ref, o_ref, *bufs):
    ibufs, obufs = bufs[0:NBUF], bufs[NBUF:2 * NBUF]
    isems, osems = bufs[2 * NBUF:3 * NBUF], bufs[3 * NBUF:4 * NBUF]
    j = pl.program_id(0)

    def in_cp(c, buf, sem):
        return pltpu.make_async_copy(
            lg_ref.at[pl.ds(c * CH4, CH4), :, :, :], buf, sem)

    def out_cp(c, buf, sem):
        return pltpu.make_async_copy(
            buf, o_ref.at[pl.ds(c * CH4, CH4), :, :, :], sem)

    @pl.when(j == 0)
    def _():
        for b in range(NBUF):
            in_cp(b, ibufs[b], isems[b]).start()

    for b in range(NBUF):
        c = j * NBUF + b
        in_cp(c, ibufs[b], isems[b]).wait()

        @pl.when(j > 0)
        def _(b=b, c=c):
            out_cp(c - NBUF, obufs[b], osems[b]).wait()

        x = ibufs[b][...]
        lab = lab_ref[...].reshape(1, B // 128, 1, 128)
        ca = ca_ref[...].reshape(1, B // 128, 1, 128)
        sa = sa_ref[...].reshape(1, B // 128, 1, 128)
        off = off_ref[...].reshape(1, B // 128, 1, 128)
        ci = lax.broadcasted_iota(jnp.int32, (CH4, 1, 8, 1), 0)
        ck = lax.broadcasted_iota(jnp.int32, (CH4, 1, 8, 1), 2)
        cls = (c * CH4 + ci) * 8 + ck
        mask = cls == lab
        t = jnp.sum(jnp.where(mask, x, 0.0), axis=0, keepdims=True)
        t = jnp.sum(t, axis=2, keepdims=True)
        root = jnp.sqrt(jnp.maximum(1.0 - t * t, 0.0))
        g = (t * ca + root * sa - off) * S
        obufs[b][...] = jnp.where(mask, g, x * S)
        out_cp(c, obufs[b], osems[b]).start()

        @pl.when(j < NSTEP - 1)
        def _(b=b, c=c):
            in_cp(c + NBUF, ibufs[b], isems[b]).start()

    @pl.when(j == NSTEP - 1)
    def _():
        for b in range(NBUF):
            out_cp(j * NBUF + b, obufs[b], osems[b]).wait()


def _tc_scale(lg4, labels2d, ca2d, sa2d, off2d):
    return pl.pallas_call(
        _tc_body,
        grid=(NSTEP,),
        in_specs=[
            pl.BlockSpec((B // 128, 128), lambda j: (0, 0)),
            pl.BlockSpec((B // 128, 128), lambda j: (0, 0)),
            pl.BlockSpec((B // 128, 128), lambda j: (0, 0)),
            pl.BlockSpec((B // 128, 128), lambda j: (0, 0)),
            pl.BlockSpec(memory_space=pl.ANY),
        ],
        out_specs=pl.BlockSpec(memory_space=pl.ANY),
        out_shape=jax.ShapeDtypeStruct((C // 8, B // 128, 8, 128),
                                       jnp.float32),
        scratch_shapes=(
            [pltpu.VMEM((CH4, B // 128, 8, 128), jnp.float32)] * (2 * NBUF)
            + [pltpu.SemaphoreType.DMA] * (2 * NBUF)
        ),
        compiler_params=pltpu.CompilerParams(
            dimension_semantics=("arbitrary",),
            vmem_limit_bytes=100 * 1024 * 1024),
    )(labels2d, ca2d, sa2d, off2d, lg4)


def kernel(logits, labels, norms, batch_mean, batch_std):
    bm16 = jnp.broadcast_to(batch_mean, (L,))
    bs16 = jnp.broadcast_to(batch_std, (L,))
    ca, sa, off, nm16, ns16 = _sc_prep(norms.reshape(B), bm16, bs16)
    lg4 = logits.T.reshape(C // 8, 8, B // 128, 128).transpose(0, 2, 1, 3)
    out4 = _tc_scale(lg4, labels.reshape(B // 128, 128),
                     ca.reshape(B // 128, 128), sa.reshape(B // 128, 128),
                     off.reshape(B // 128, 128))
    out = out4.transpose(0, 2, 1, 3).reshape(C, B).T
    return out, nm16[:1], ns16[:1]
